# baseline (speedup 1.0000x reference)
import functools

import jax
import jax.numpy as jnp
from jax import lax
from jax.experimental import pallas as pl
from jax.experimental.pallas import tpu as pltpu


def _mha_kernel(x_ref, w_ref, bqkv_ref, wo_ref, ob_ref,
                out_ref, attn_ref, qs_ref, ks_ref, vs_ref, *,
                g, dk, nj, n_items):
    n = pl.program_id(0)
    gd = g * dk
    D = x_ref.shape[2]

    item_p = jnp.minimum(n, n_items - 1)
    jn = lax.rem(item_p, nj)
    wslot = lax.rem(n, 2)
    item_c = jnp.maximum(n - 1, 0)
    jc = lax.rem(item_c, nj)
    rslot = lax.rem(n + 1, 2)

    x = x_ref[0].astype(jnp.bfloat16)

    def proj_group(base, dst_ref):
        acc = jnp.dot(x, w_ref[:, pl.ds(base, gd)],
                      preferred_element_type=jnp.float32)
        dst_ref[wslot] = (acc + bqkv_ref[:, pl.ds(base, gd)]).astype(jnp.bfloat16)

    proj_group(jn * gd, qs_ref)
    proj_group(D + jn * gd, ks_ref)
    proj_group(2 * D + jn * gd, vs_ref)

    q = qs_ref[rslot]
    k = ks_ref[rslot]
    v = vs_ref[rslot]

    ctx_parts = []
    for h in range(g):
        sl = slice(h * dk, (h + 1) * dk)
        s = lax.dot_general(q[:, sl], k[:, sl], (((1,), (1,)), ((), ())),
                            preferred_element_type=jnp.float32)
        row_max = jnp.max(s, axis=-1, keepdims=True)
        denom = jnp.sum(jnp.exp(s - row_max), axis=-1, keepdims=True)
        p = jnp.exp(s - (row_max + jnp.log(denom)))
        attn_ref[0, h] = p
        ctx_parts.append(jnp.dot(p.astype(jnp.bfloat16), v[:, sl],
                                 preferred_element_type=jnp.float32))

    ctx = jnp.concatenate(ctx_parts, axis=1).astype(jnp.bfloat16)
    wo = wo_ref[pl.ds(jc * gd, gd), :]
    partial = jnp.dot(ctx, wo, preferred_element_type=jnp.float32)

    @pl.when(jc == 0)
    def _init():
        out_ref[0] = partial + ob_ref[...]

    @pl.when(jc != 0)
    def _acc():
        out_ref[0] += partial


def kernel(x, qkv_wt, qkv_b, o_wt, o_b):
    bs, L, D = x.shape
    dk = 64
    nh = D // dk
    g = 8
    nj = nh // g
    gd = g * dk
    n_items = bs * nj

    b2 = qkv_b.reshape(1, 3 * D).astype(jnp.float32)
    ob2 = o_b.reshape(1, D).astype(jnp.float32)

    def item_proj_b(n):
        return lax.div(jnp.minimum(n, n_items - 1), nj)

    def item_cur(n):
        return jnp.maximum(n - 1, 0)

    out, attn = pl.pallas_call(
        functools.partial(_mha_kernel, g=g, dk=dk, nj=nj, n_items=n_items),
        out_shape=(
            jax.ShapeDtypeStruct((bs, L, D), jnp.float32),
            jax.ShapeDtypeStruct((bs, nh, L, L), jnp.float32),
        ),
        grid=(n_items + 1,),
        in_specs=[
            pl.BlockSpec((1, L, D), lambda n: (item_proj_b(n), 0, 0)),
            pl.BlockSpec((D, 3 * D), lambda n: (0, 0)),
            pl.BlockSpec((1, 3 * D), lambda n: (0, 0)),
            pl.BlockSpec((D, D), lambda n: (0, 0)),
            pl.BlockSpec((1, D), lambda n: (0, 0)),
        ],
        out_specs=(
            pl.BlockSpec((1, L, D), lambda n: (lax.div(item_cur(n), nj), 0, 0)),
            pl.BlockSpec((1, g, L, L),
                         lambda n: (lax.div(item_cur(n), nj),
                                    lax.rem(item_cur(n), nj), 0, 0)),
        ),
        scratch_shapes=[
            pltpu.VMEM((2, L, gd), jnp.bfloat16),
            pltpu.VMEM((2, L, gd), jnp.bfloat16),
            pltpu.VMEM((2, L, gd), jnp.bfloat16),
        ],
        compiler_params=pltpu.CompilerParams(
            dimension_semantics=("arbitrary",),
            vmem_limit_bytes=56 * 1024 * 1024,
        ),
    )(x, qkv_wt, b2, o_wt, ob2)
    return out, attn


# speedup vs baseline: 1.0564x; 1.0564x over previous
import functools

import jax
import jax.numpy as jnp
from jax import lax
from jax.experimental import pallas as pl
from jax.experimental.pallas import tpu as pltpu


def _mha_kernel(x_ref, w_ref, bqkv_ref, wo_ref, ob_ref,
                out_ref, attn_ref, ctx_s_ref, *, g, dk):
    j = pl.program_id(1)
    gd = g * dk
    D = x_ref.shape[2]

    x = x_ref[0].astype(jnp.bfloat16)

    def proj_group(base):
        acc = jnp.dot(x, w_ref[:, pl.ds(base, gd)],
                      preferred_element_type=jnp.float32)
        return (acc + bqkv_ref[:, pl.ds(base, gd)]).astype(jnp.bfloat16)

    q = proj_group(j * gd)
    k = proj_group(D + j * gd)
    v = proj_group(2 * D + j * gd)

    ctx_parts = []
    for h in range(g):
        sl = slice(h * dk, (h + 1) * dk)
        s = lax.dot_general(q[:, sl], k[:, sl], (((1,), (1,)), ((), ())),
                            preferred_element_type=jnp.float32)
        row_max = jnp.max(s, axis=-1, keepdims=True)
        denom = jnp.sum(jnp.exp(s - row_max), axis=-1, keepdims=True)
        p = jnp.exp(s - (row_max + jnp.log(denom)))
        attn_ref[0, h] = p
        ctx_parts.append(jnp.dot(p.astype(jnp.bfloat16), v[:, sl],
                                 preferred_element_type=jnp.float32))

    ctx = jnp.concatenate(ctx_parts, axis=1).astype(jnp.bfloat16)

    @pl.when(j == 0)
    def _stage():
        ctx_s_ref[...] = ctx

    @pl.when(j != 0)
    def _project_out():
        ctx_full = jnp.concatenate([ctx_s_ref[...], ctx], axis=1)
        out_ref[0] = (jnp.dot(ctx_full, wo_ref[...],
                              preferred_element_type=jnp.float32)
                      + ob_ref[...])


def kernel(x, qkv_wt, qkv_b, o_wt, o_b):
    bs, L, D = x.shape
    dk = 64
    nh = D // dk
    g = 8
    nj = nh // g
    gd = g * dk

    b2 = qkv_b.reshape(1, 3 * D).astype(jnp.float32)
    ob2 = o_b.reshape(1, D).astype(jnp.float32)

    out, attn = pl.pallas_call(
        functools.partial(_mha_kernel, g=g, dk=dk),
        out_shape=(
            jax.ShapeDtypeStruct((bs, L, D), jnp.float32),
            jax.ShapeDtypeStruct((bs, nh, L, L), jnp.float32),
        ),
        grid=(bs, nj),
        in_specs=[
            pl.BlockSpec((1, L, D), lambda b, j: (b, 0, 0)),
            pl.BlockSpec((D, 3 * D), lambda b, j: (0, 0)),
            pl.BlockSpec((1, 3 * D), lambda b, j: (0, 0)),
            pl.BlockSpec((D, D), lambda b, j: (0, 0)),
            pl.BlockSpec((1, D), lambda b, j: (0, 0)),
        ],
        out_specs=(
            pl.BlockSpec((1, L, D), lambda b, j: (b, 0, 0)),
            pl.BlockSpec((1, g, L, L), lambda b, j: (b, j, 0, 0)),
        ),
        scratch_shapes=[pltpu.VMEM((L, gd), jnp.bfloat16)],
        compiler_params=pltpu.CompilerParams(
            dimension_semantics=("parallel", "arbitrary"),
            vmem_limit_bytes=56 * 1024 * 1024,
        ),
    )(x, qkv_wt, b2, o_wt, ob2)
    return out, attn


# speedup vs baseline: 1.1272x; 1.0670x over previous
import functools

import jax
import jax.numpy as jnp
from jax import lax
from jax.experimental import pallas as pl
from jax.experimental.pallas import tpu as pltpu


def _mha_kernel(x_ref, w_ref, bqkv_ref, wo_ref, ob_ref,
                out_ref, attn_ref, ctx_s_ref, *, g, dk):
    j = pl.program_id(1)
    gd = g * dk
    D = x_ref.shape[2]

    x = x_ref[0].astype(jnp.bfloat16)

    def proj_group(base):
        acc = jnp.dot(x, w_ref[:, pl.ds(base, gd)],
                      preferred_element_type=jnp.float32)
        return (acc + bqkv_ref[:, pl.ds(base, gd)]).astype(jnp.bfloat16)

    q = proj_group(j * gd)
    k = proj_group(D + j * gd)
    v = proj_group(2 * D + j * gd)

    ctx_parts = []
    for h in range(g):
        sl = slice(h * dk, (h + 1) * dk)
        s = lax.dot_general(q[:, sl], k[:, sl], (((1,), (1,)), ((), ())),
                            preferred_element_type=jnp.float32)
        row_max = jnp.max(s, axis=-1, keepdims=True)
        eb = jnp.exp(s - row_max).astype(jnp.bfloat16)
        denom = jnp.sum(eb.astype(jnp.float32), axis=-1, keepdims=True)
        recip = 1.0 / denom
        attn_ref[0, h] = eb.astype(jnp.float32) * recip
        ctx_un = jnp.dot(eb, v[:, sl], preferred_element_type=jnp.float32)
        ctx_parts.append(ctx_un * recip)

    ctx = jnp.concatenate(ctx_parts, axis=1).astype(jnp.bfloat16)

    @pl.when(j == 0)
    def _stage():
        ctx_s_ref[...] = ctx

    @pl.when(j != 0)
    def _project_out():
        ctx_full = jnp.concatenate([ctx_s_ref[...], ctx], axis=1)
        out_ref[0] = (jnp.dot(ctx_full, wo_ref[...],
                              preferred_element_type=jnp.float32)
                      + ob_ref[...])


def kernel(x, qkv_wt, qkv_b, o_wt, o_b):
    bs, L, D = x.shape
    dk = 64
    nh = D // dk
    g = 8
    nj = nh // g
    gd = g * dk

    b2 = qkv_b.reshape(1, 3 * D).astype(jnp.float32)
    ob2 = o_b.reshape(1, D).astype(jnp.float32)

    out, attn = pl.pallas_call(
        functools.partial(_mha_kernel, g=g, dk=dk),
        out_shape=(
            jax.ShapeDtypeStruct((bs, L, D), jnp.float32),
            jax.ShapeDtypeStruct((bs, nh, L, L), jnp.float32),
        ),
        grid=(bs, nj),
        in_specs=[
            pl.BlockSpec((1, L, D), lambda b, j: (b, 0, 0)),
            pl.BlockSpec((D, 3 * D), lambda b, j: (0, 0)),
            pl.BlockSpec((1, 3 * D), lambda b, j: (0, 0)),
            pl.BlockSpec((D, D), lambda b, j: (0, 0)),
            pl.BlockSpec((1, D), lambda b, j: (0, 0)),
        ],
        out_specs=(
            pl.BlockSpec((1, L, D), lambda b, j: (b, 0, 0)),
            pl.BlockSpec((1, g, L, L), lambda b, j: (b, j, 0, 0)),
        ),
        scratch_shapes=[pltpu.VMEM((L, gd), jnp.bfloat16)],
        compiler_params=pltpu.CompilerParams(
            dimension_semantics=("parallel", "arbitrary"),
            vmem_limit_bytes=56 * 1024 * 1024,
        ),
    )(x, qkv_wt, b2, o_wt, ob2)
    return out, attn


# speedup vs baseline: 1.1697x; 1.0377x over previous
import functools

import jax
import jax.numpy as jnp
from jax import lax
from jax.experimental import pallas as pl
from jax.experimental.pallas import tpu as pltpu


def _mha_kernel(x_ref, w_ref, bqkv_ref, wo_ref, ob_ref,
                out_ref, attn_ref, ctx_s_ref, *, g, dk):
    j = pl.program_id(1)
    gd = g * dk
    D = x_ref.shape[2]

    x = x_ref[0].astype(jnp.bfloat16)

    def proj_group(base):
        acc = jnp.dot(x, w_ref[:, pl.ds(base, gd)],
                      preferred_element_type=jnp.float32)
        return (acc + bqkv_ref[:, pl.ds(base, gd)]).astype(jnp.bfloat16)

    q = proj_group(j * gd)
    k = proj_group(D + j * gd)
    v = proj_group(2 * D + j * gd)

    def qk(h):
        sl = slice(h * dk, (h + 1) * dk)
        return lax.dot_general(q[:, sl], k[:, sl], (((1,), (1,)), ((), ())),
                               preferred_element_type=jnp.float32)

    ctx_parts = []
    s = qk(0)
    for h in range(g):
        row_max = jnp.max(s, axis=-1, keepdims=True)
        eb = jnp.exp(s - row_max).astype(jnp.bfloat16)
        if h + 1 < g:
            s = qk(h + 1)
        denom = jnp.sum(eb.astype(jnp.float32), axis=-1, keepdims=True)
        recip = 1.0 / denom
        attn_ref[0, h] = eb.astype(jnp.float32) * recip
        ctx_un = jnp.dot(eb, v[:, h * dk:(h + 1) * dk],
                         preferred_element_type=jnp.float32)
        ctx_parts.append(ctx_un * recip)

    ctx = jnp.concatenate(ctx_parts, axis=1).astype(jnp.bfloat16)

    @pl.when(j == 0)
    def _stage():
        ctx_s_ref[...] = ctx

    @pl.when(j != 0)
    def _project_out():
        ctx_full = jnp.concatenate([ctx_s_ref[...], ctx], axis=1)
        out_ref[0] = (jnp.dot(ctx_full, wo_ref[...],
                              preferred_element_type=jnp.float32)
                      + ob_ref[...])


def kernel(x, qkv_wt, qkv_b, o_wt, o_b):
    bs, L, D = x.shape
    dk = 64
    nh = D // dk
    g = 8
    nj = nh // g
    gd = g * dk

    b2 = qkv_b.reshape(1, 3 * D).astype(jnp.float32)
    ob2 = o_b.reshape(1, D).astype(jnp.float32)

    out, attn = pl.pallas_call(
        functools.partial(_mha_kernel, g=g, dk=dk),
        out_shape=(
            jax.ShapeDtypeStruct((bs, L, D), jnp.float32),
            jax.ShapeDtypeStruct((bs, nh, L, L), jnp.float32),
        ),
        grid=(bs, nj),
        in_specs=[
            pl.BlockSpec((1, L, D), lambda b, j: (b, 0, 0)),
            pl.BlockSpec((D, 3 * D), lambda b, j: (0, 0)),
            pl.BlockSpec((1, 3 * D), lambda b, j: (0, 0)),
            pl.BlockSpec((D, D), lambda b, j: (0, 0)),
            pl.BlockSpec((1, D), lambda b, j: (0, 0)),
        ],
        out_specs=(
            pl.BlockSpec((1, L, D), lambda b, j: (b, 0, 0)),
            pl.BlockSpec((1, g, L, L), lambda b, j: (b, j, 0, 0)),
        ),
        scratch_shapes=[pltpu.VMEM((L, gd), jnp.bfloat16)],
        compiler_params=pltpu.CompilerParams(
            dimension_semantics=("parallel", "arbitrary"),
            vmem_limit_bytes=56 * 1024 * 1024,
        ),
    )(x, qkv_wt, b2, o_wt, ob2)
    return out, attn


# speedup vs baseline: 1.1858x; 1.0137x over previous
import functools

import jax
import jax.numpy as jnp
from jax import lax
from jax.experimental import pallas as pl
from jax.experimental.pallas import tpu as pltpu


def _mha_kernel(x_ref, w_ref, bqkv_ref, wo_ref, ob_ref,
                out_ref, attn_ref, ctx_s_ref, *, g, dk):
    j = pl.program_id(1)
    gd = g * dk
    D = x_ref.shape[2]

    x = x_ref[0].astype(jnp.bfloat16)

    def proj_group(base):
        acc = jnp.dot(x, w_ref[:, pl.ds(base, gd)],
                      preferred_element_type=jnp.float32)
        return (acc + bqkv_ref[:, pl.ds(base, gd)]).astype(jnp.bfloat16)

    q = proj_group(j * gd)
    k = proj_group(D + j * gd)
    v = proj_group(2 * D + j * gd)

    def qk(h):
        sl = slice(h * dk, (h + 1) * dk)
        return lax.dot_general(q[:, sl], k[:, sl], (((1,), (1,)), ((), ())),
                               preferred_element_type=jnp.float32)

    ctx_parts = []
    s = qk(0)
    for h in range(g):
        row_max = jnp.max(s, axis=-1, keepdims=True)
        s_cur = s
        if h + 1 < g:
            s = qk(h + 1)
        eb = jnp.exp(s_cur - row_max).astype(jnp.bfloat16)
        ctx_un = jnp.dot(eb, v[:, h * dk:(h + 1) * dk],
                         preferred_element_type=jnp.float32)
        denom = jnp.sum(eb.astype(jnp.float32), axis=-1, keepdims=True)
        recip = 1.0 / denom
        attn_ref[0, h] = eb.astype(jnp.float32) * recip
        ctx_parts.append(ctx_un * recip)

    ctx = jnp.concatenate(ctx_parts, axis=1).astype(jnp.bfloat16)

    @pl.when(j == 0)
    def _stage():
        ctx_s_ref[...] = ctx

    @pl.when(j != 0)
    def _project_out():
        ctx_full = jnp.concatenate([ctx_s_ref[...], ctx], axis=1)
        out_ref[0] = (jnp.dot(ctx_full, wo_ref[...],
                              preferred_element_type=jnp.float32)
                      + ob_ref[...])


def kernel(x, qkv_wt, qkv_b, o_wt, o_b):
    bs, L, D = x.shape
    dk = 64
    nh = D // dk
    g = 8
    nj = nh // g
    gd = g * dk

    b2 = qkv_b.reshape(1, 3 * D).astype(jnp.float32)
    ob2 = o_b.reshape(1, D).astype(jnp.float32)

    out, attn = pl.pallas_call(
        functools.partial(_mha_kernel, g=g, dk=dk),
        out_shape=(
            jax.ShapeDtypeStruct((bs, L, D), jnp.float32),
            jax.ShapeDtypeStruct((bs, nh, L, L), jnp.float32),
        ),
        grid=(bs, nj),
        in_specs=[
            pl.BlockSpec((1, L, D), lambda b, j: (b, 0, 0)),
            pl.BlockSpec((D, 3 * D), lambda b, j: (0, 0)),
            pl.BlockSpec((1, 3 * D), lambda b, j: (0, 0)),
            pl.BlockSpec((D, D), lambda b, j: (0, 0)),
            pl.BlockSpec((1, D), lambda b, j: (0, 0)),
        ],
        out_specs=(
            pl.BlockSpec((1, L, D), lambda b, j: (b, 0, 0)),
            pl.BlockSpec((1, g, L, L), lambda b, j: (b, j, 0, 0)),
        ),
        scratch_shapes=[pltpu.VMEM((L, gd), jnp.bfloat16)],
        compiler_params=pltpu.CompilerParams(
            dimension_semantics=("parallel", "arbitrary"),
            vmem_limit_bytes=56 * 1024 * 1024,
        ),
    )(x, qkv_wt, b2, o_wt, ob2)
    return out, attn
